# baseline (speedup 1.0000x reference)
import functools

import jax
import jax.numpy as jnp
from jax import lax
from jax.experimental import pallas as pl
from jax.experimental.pallas import tpu as pltpu
from jax.experimental.pallas import tpu_sc as plsc

B = 16384
S = 8
D = 128
NC = 2
NS = 16
NW = NC * NS
CHUNK = 256
TB = 512
GRID = B // TB


def _sc_gather(in_embed, out_embed, idx_inp, idx_outp, idx_noise, idx_cpn):
    mesh = plsc.VectorSubcoreMesh(core_axis_name="c", subcore_axis_name="s")

    @functools.partial(
        pl.kernel,
        mesh=mesh,
        out_type=[
            jax.ShapeDtypeStruct((B, D), jnp.float32),
            jax.ShapeDtypeStruct((B, D), jnp.float32),
            jax.ShapeDtypeStruct((B * S, D), jnp.float32),
            jax.ShapeDtypeStruct((B * S, D), jnp.float32),
        ],
        scratch_types=[
            pltpu.VMEM((CHUNK,), jnp.int32),
            pltpu.VMEM((CHUNK, D), jnp.float32),
            pltpu.SemaphoreType.DMA,
        ],
    )
    def k(in_hbm, out_hbm, ii_hbm, io_hbm, inoise_hbm, icpn_hbm,
          r_inp, r_outp, r_noise, r_cpn, idx_v, rows_v, sem):
        wid = lax.axis_index("s") * NC + lax.axis_index("c")

        def gather_arr(tbl, idx_hbm, dst_hbm, n_total):
            n_per_w = n_total // NW
            nchunks = n_per_w // CHUNK
            wbase = wid * n_per_w

            def body(c, carry):
                off = wbase + c * CHUNK
                pltpu.sync_copy(idx_hbm.at[pl.ds(off, CHUNK)], idx_v)
                pltpu.async_copy(tbl.at[idx_v], rows_v, sem).wait()
                pltpu.sync_copy(rows_v, dst_hbm.at[pl.ds(off, CHUNK), :])
                return carry

            lax.fori_loop(0, nchunks, body, 0)

        gather_arr(in_hbm, ii_hbm, r_inp, B)
        gather_arr(out_hbm, io_hbm, r_outp, B)
        gather_arr(in_hbm, inoise_hbm, r_noise, B * S)
        gather_arr(out_hbm, icpn_hbm, r_cpn, B * S)

    return k(in_embed, out_embed, idx_inp, idx_outp, idx_noise, idx_cpn)


def _tc_loss_body(types_ref, inp_ref, outp_ref, noise_ref, cpn_ref, w_ref,
                  out_ref):
    i = pl.program_id(0)
    t = types_ref[0, 0, :]
    inp = inp_ref[...]
    outp = outp_ref[...]
    nrows = noise_ref[...]
    crows = cpn_ref[...]
    w3 = w_ref[...]

    tb = t[:, None]
    w = jnp.where(tb == 0, w3[0][None, :],
                  jnp.where(tb == 1, w3[1][None, :], w3[2][None, :]))
    q = outp * w
    p = inp * w
    z = jnp.sum(inp * q, axis=1)
    zu = -jnp.sum(nrows * q[:, None, :], axis=2)
    zv = -jnp.sum(crows * p[:, None, :], axis=2)

    ls = jax.nn.log_sigmoid
    pos = 2.0 * jnp.sum(ls(z)) + jnp.sum(ls(zu)) + jnp.sum(ls(zv))
    reg = (jnp.sum(inp * inp) + jnp.sum(outp * outp)
           + jnp.sum(nrows * nrows) + jnp.sum(crows * crows)
           + jnp.sum(w * w))
    partial = pos - reg

    @pl.when(i == 0)
    def _():
        out_ref[0, 0] = 0.0

    out_ref[0, 0] += partial


def _tc_loss(types3, inp, outp, noise3, cpn3, edge_weights):
    return pl.pallas_call(
        _tc_loss_body,
        grid=(GRID,),
        in_specs=[
            pl.BlockSpec((1, 1, TB), lambda i: (i, 0, 0)),
            pl.BlockSpec((TB, D), lambda i: (i, 0)),
            pl.BlockSpec((TB, D), lambda i: (i, 0)),
            pl.BlockSpec((TB, S, D), lambda i: (i, 0, 0)),
            pl.BlockSpec((TB, S, D), lambda i: (i, 0, 0)),
            pl.BlockSpec((3, D), lambda i: (0, 0)),
        ],
        out_specs=pl.BlockSpec(memory_space=pltpu.SMEM),
        out_shape=jax.ShapeDtypeStruct((1, 1), jnp.float32),
    )(types3, inp, outp, noise3, cpn3, edge_weights)


def kernel(input_labels, out_labels, noise_u, cp_noise_v, in_embed, out_embed,
           edge_weights):
    types = input_labels[:, 0]
    rows_inp, rows_outp, rows_noise, rows_cpn = _sc_gather(
        in_embed, out_embed,
        input_labels[:, 1], out_labels[:, 1],
        noise_u.reshape(-1), cp_noise_v.reshape(-1))
    total = _tc_loss(
        types.reshape(GRID, 1, TB),
        rows_inp, rows_outp,
        rows_noise.reshape(B, S, D), rows_cpn.reshape(B, S, D),
        edge_weights)
    return -total[0, 0] / (2.0 * B)
